# baseline (speedup 1.0000x reference)
import functools

import jax
import jax.numpy as jnp
from jax import lax
from jax.experimental import pallas as pl
from jax.experimental.pallas import tpu as pltpu
from jax.experimental.pallas import tpu_sc as plsc

V = 1_000_000
B = 128
R, C = 1000, 1000
BR = 40
NB = R // BR

_mesh = plsc.VectorSubcoreMesh(
    core_axis_name="c", subcore_axis_name="s", num_cores=1, num_subcores=16
)


@functools.partial(
    pl.kernel,
    out_type=jax.ShapeDtypeStruct((B,), jnp.float32),
    mesh=_mesh,
    scratch_types=[
        pltpu.VMEM((B,), jnp.int32),
        pltpu.VMEM((B,), jnp.float32),
        pltpu.SemaphoreType.DMA,
    ],
)
def _sc_gather(logits_hbm, value_hbm, out_g, idx_v, g_v, sem):
    sid = lax.axis_index("s")
    cid = lax.axis_index("c")

    @pl.when((sid == 0) & (cid == 0))
    def _():
        pltpu.sync_copy(value_hbm, idx_v)
        pltpu.async_copy(logits_hbm.at[idx_v], g_v, sem).wait()
        pltpu.sync_copy(g_v, out_g)


def _partials_body(x_ref, bm_ref, bs_ref):
    x = x_ref[...]
    bm = jnp.max(x)
    bm_ref[...] = jnp.full((1, 1, B), bm, jnp.float32)
    bs_ref[...] = jnp.full((1, 1, B), jnp.sum(jnp.exp(x - bm)), jnp.float32)


def _tc_partials(x2d):
    return pl.pallas_call(
        _partials_body,
        grid=(NB,),
        in_specs=[pl.BlockSpec((BR, C), lambda i: (i, 0))],
        out_specs=[
            pl.BlockSpec((1, 1, B), lambda i: (i, 0, 0)),
            pl.BlockSpec((1, 1, B), lambda i: (i, 0, 0)),
        ],
        out_shape=[
            jax.ShapeDtypeStruct((NB, 1, B), jnp.float32),
            jax.ShapeDtypeStruct((NB, 1, B), jnp.float32),
        ],
    )(x2d)


def _combine_body(bm_ref, bs_ref, g_ref, o_ref):
    bm = bm_ref[...]
    gmax = jnp.max(bm)
    total = jnp.sum(bs_ref[...] * jnp.exp(bm - gmax)) * (1.0 / B)
    o_ref[...] = g_ref[...] - gmax - jnp.log(total)


def _tc_combine(bm, bs, g):
    return pl.pallas_call(
        _combine_body,
        out_shape=jax.ShapeDtypeStruct((1, B), jnp.float32),
    )(bm, bs, g)


def kernel(logits, value):
    g = _sc_gather(logits, value)
    bm, bs = _tc_partials(logits.reshape(R, C))
    out = _tc_combine(bm.reshape(NB, B), bs.reshape(NB, B), g.reshape(1, B))
    return out.reshape(B)
